# baseline (device time: 84506 ns/iter reference)
import jax
import jax.numpy as jnp
from jax import lax
from jax.experimental import pallas as pl
from jax.experimental.pallas import tpu as pltpu

N_DEV = 8
T = 1024
V_PER = 8192
D = 1024
R = 8
CH = 32
REG = 384


def kernel(ids, E):
    owner = lax.shift_right_logical(ids, 13)
    perm = jnp.argsort(owner, stable=True).astype(jnp.int32)
    onehot = (
        owner[:, None] == jnp.arange(N_DEV, dtype=ids.dtype)[None, :]
    ).astype(jnp.int32)
    ranks2d = jnp.cumsum(onehot, axis=0) - onehot
    rank = jnp.take_along_axis(ranks2d, owner[:, None], axis=1)[:, 0]
    rank = rank.astype(jnp.int32)
    counts = jnp.sum(onehot, axis=0, dtype=jnp.int32)
    starts = jnp.concatenate(
        [jnp.zeros((1,), jnp.int32), jnp.cumsum(counts)[:-1].astype(jnp.int32)]
    )

    def body(perm_ref, ids_ref, rank_ref, cnt_ref, start_ref, e_ref,
             out_ref, gat_ref, stage_ref, recv_ref, gsem, send_sem,
             recv_sem):
        my = lax.axis_index("i")
        base = my * V_PER
        my_start = start_ref[my]
        own_cnt = cnt_ref[my]
        my_chunks = (own_cnt + CH - 1) // CH

        def g_body(k, _):
            lid = ids_ref[perm_ref[my_start + k]] - base
            pltpu.make_async_copy(
                e_ref.at[pl.ds(lid * R, R), :],
                gat_ref.at[pl.ds(k * R, R), :],
                gsem,
            ).start()
            return _

        lax.fori_loop(0, own_cnt, g_body, 0)

        def tc_body(s, tot):
            return tot + (cnt_ref[s] + CH - 1) // CH

        total_chunks = lax.fori_loop(0, N_DEV, tc_body, jnp.int32(0))

        barrier_sem = pltpu.get_barrier_semaphore()
        for j in range(1, N_DEV):
            pl.semaphore_signal(
                barrier_sem,
                1,
                device_id=(lax.rem(my + j, N_DEV),),
                device_id_type=pl.DeviceIdType.MESH,
            )
        pl.semaphore_wait(barrier_sem, N_DEV - 1)

        def gw_body(_, x):
            pltpu.make_async_copy(
                e_ref.at[pl.ds(0, R), :], gat_ref.at[pl.ds(0, R), :], gsem
            ).wait()
            return x

        lax.fori_loop(0, own_cnt, gw_body, 0)
        stage_ref[:, :] = gat_ref[:, :].astype(jnp.bfloat16)

        def c_body(c, _):
            src = stage_ref.at[pl.ds(c * CH * R, CH * R), :]
            dst = recv_ref.at[pl.ds((my * REG + c * CH) * R, CH * R), :]
            for j in range(1, N_DEV):
                pltpu.make_async_remote_copy(
                    src_ref=src,
                    dst_ref=dst,
                    send_sem=send_sem,
                    recv_sem=recv_sem,
                    device_id=(lax.rem(my + j, N_DEV),),
                    device_id_type=pl.DeviceIdType.MESH,
                ).start()
            pltpu.make_async_copy(src, dst, recv_sem).start()
            return _

        lax.fori_loop(0, my_chunks, c_body, 0)

        def w_body(_, x):
            pltpu.make_async_remote_copy(
                src_ref=stage_ref.at[pl.ds(0, (N_DEV - 1) * CH * R), :],
                dst_ref=recv_ref.at[pl.ds(0, (N_DEV - 1) * CH * R), :],
                send_sem=send_sem,
                recv_sem=recv_sem,
                device_id=(my,),
                device_id_type=pl.DeviceIdType.MESH,
            ).wait_send()
            return x

        lax.fori_loop(0, my_chunks, w_body, 0)

        def r_body(_, x):
            pltpu.make_async_remote_copy(
                src_ref=stage_ref.at[pl.ds(0, CH * R), :],
                dst_ref=recv_ref.at[pl.ds(0, CH * R), :],
                send_sem=send_sem,
                recv_sem=recv_sem,
                device_id=(my,),
                device_id_type=pl.DeviceIdType.MESH,
            ).wait_recv()
            return x

        lax.fori_loop(0, total_chunks, r_body, 0)

        def u_body(t, _):
            o = lax.shift_right_logical(ids_ref[t], 13)
            k = rank_ref[t]
            out_ref[pl.ds(t * R, R), :] = recv_ref[
                pl.ds((o * REG + k) * R, R), :
            ]
            return _

        lax.fori_loop(0, T, u_body, 0, unroll=8)

    out = pl.pallas_call(
        body,
        out_shape=jax.ShapeDtypeStruct((T * R, 128), jnp.bfloat16),
        in_specs=[
            pl.BlockSpec(memory_space=pltpu.SMEM),
            pl.BlockSpec(memory_space=pltpu.SMEM),
            pl.BlockSpec(memory_space=pltpu.SMEM),
            pl.BlockSpec(memory_space=pltpu.SMEM),
            pl.BlockSpec(memory_space=pltpu.SMEM),
            pl.BlockSpec(memory_space=pl.ANY),
        ],
        out_specs=pl.BlockSpec(memory_space=pltpu.VMEM),
        scratch_shapes=[
            pltpu.VMEM((T * R, 128), jnp.float32),
            pltpu.VMEM((T * R, 128), jnp.bfloat16),
            pltpu.VMEM((N_DEV * REG * R, 128), jnp.bfloat16),
            pltpu.SemaphoreType.DMA,
            pltpu.SemaphoreType.DMA,
            pltpu.SemaphoreType.DMA,
        ],
        compiler_params=pltpu.CompilerParams(collective_id=0),
    )(perm, ids, rank, counts, starts, E.reshape(V_PER * R, 128))
    return out.reshape(T, D)


# device time: 66366 ns/iter; 1.2733x vs baseline; 1.2733x over previous
import jax
import jax.numpy as jnp
from jax import lax
from jax.experimental import pallas as pl
from jax.experimental.pallas import tpu as pltpu

N_DEV = 8
T = 1024
V_PER = 8192
D = 1024
R = 8
SEND_WINDOW = 16


def kernel(ids, E):
    owner = lax.shift_right_logical(ids, 13)
    perm = jnp.argsort(owner, stable=True).astype(jnp.int32)
    counts = jnp.sum(
        owner[:, None] == jnp.arange(N_DEV, dtype=ids.dtype)[None, :],
        axis=0,
        dtype=jnp.int32,
    )
    starts = jnp.concatenate(
        [jnp.zeros((1,), jnp.int32), jnp.cumsum(counts)[:-1].astype(jnp.int32)]
    )

    def body(perm_ref, ids_ref, cnt_ref, start_ref, e_ref, out_ref,
             gat_ref, stage_ref, xb_ref, gsem, send_sem, recv_sem):
        my = lax.axis_index("i")
        base = my * V_PER
        my_start = start_ref[my]
        own_cnt = cnt_ref[my]

        def g_body(k, _):
            lid = ids_ref[perm_ref[my_start + k]] - base
            pltpu.make_async_copy(
                e_ref.at[pl.ds(lid * R, R), :],
                gat_ref.at[pl.ds(k * R, R), :],
                gsem,
            ).start()
            return _

        lax.fori_loop(0, own_cnt, g_body, 0)

        barrier_sem = pltpu.get_barrier_semaphore()
        for j in range(1, N_DEV):
            pl.semaphore_signal(
                barrier_sem,
                1,
                device_id=(lax.rem(my + j, N_DEV),),
                device_id_type=pl.DeviceIdType.MESH,
            )
        pl.semaphore_wait(barrier_sem, N_DEV - 1)

        def gw_body(_, x):
            pltpu.make_async_copy(
                e_ref.at[pl.ds(0, R), :], gat_ref.at[pl.ds(0, R), :], gsem
            ).wait()
            return x

        lax.fori_loop(0, own_cnt, gw_body, 0)
        stage_ref[:, :] = gat_ref[:, :].astype(jnp.bfloat16)

        def s_body(k, carry):
            t = perm_ref[my_start + k]
            src = stage_ref.at[pl.ds(k * R, R), :]
            for j in range(1, N_DEV):
                pltpu.make_async_remote_copy(
                    src_ref=src,
                    dst_ref=xb_ref.at[pl.ds(t * R, R), :],
                    send_sem=send_sem,
                    recv_sem=recv_sem,
                    device_id=(lax.rem(my + j, N_DEV),),
                    device_id_type=pl.DeviceIdType.MESH,
                ).start()
            pltpu.make_async_copy(
                src, xb_ref.at[pl.ds(t * R, R), :], recv_sem
            ).start()

            @pl.when(k >= SEND_WINDOW)
            def _():
                pltpu.make_async_remote_copy(
                    src_ref=stage_ref.at[pl.ds(0, (N_DEV - 1) * R), :],
                    dst_ref=xb_ref.at[pl.ds(0, (N_DEV - 1) * R), :],
                    send_sem=send_sem,
                    recv_sem=recv_sem,
                    device_id=(my,),
                    device_id_type=pl.DeviceIdType.MESH,
                ).wait_send()

            return carry

        lax.fori_loop(0, own_cnt, s_body, 0)

        def w_body(_, x):
            pltpu.make_async_remote_copy(
                src_ref=stage_ref.at[pl.ds(0, (N_DEV - 1) * R), :],
                dst_ref=xb_ref.at[pl.ds(0, (N_DEV - 1) * R), :],
                send_sem=send_sem,
                recv_sem=recv_sem,
                device_id=(my,),
                device_id_type=pl.DeviceIdType.MESH,
            ).wait_send()
            return x

        lax.fori_loop(0, jnp.minimum(own_cnt, SEND_WINDOW), w_body, 0)

        pltpu.make_async_remote_copy(
            src_ref=stage_ref.at[:, :],
            dst_ref=xb_ref.at[:, :],
            send_sem=send_sem,
            recv_sem=recv_sem,
            device_id=(my,),
            device_id_type=pl.DeviceIdType.MESH,
        ).wait_recv()

        out_ref[:, :] = xb_ref[:, :]

    out = pl.pallas_call(
        body,
        out_shape=jax.ShapeDtypeStruct((T * R, 128), jnp.bfloat16),
        in_specs=[
            pl.BlockSpec(memory_space=pltpu.SMEM),
            pl.BlockSpec(memory_space=pltpu.SMEM),
            pl.BlockSpec(memory_space=pltpu.SMEM),
            pl.BlockSpec(memory_space=pltpu.SMEM),
            pl.BlockSpec(memory_space=pl.ANY),
        ],
        out_specs=pl.BlockSpec(memory_space=pltpu.VMEM),
        scratch_shapes=[
            pltpu.VMEM((T * R, 128), jnp.float32),
            pltpu.VMEM((T * R, 128), jnp.bfloat16),
            pltpu.VMEM((T * R, 128), jnp.bfloat16),
            pltpu.SemaphoreType.DMA,
            pltpu.SemaphoreType.DMA,
            pltpu.SemaphoreType.DMA,
        ],
        compiler_params=pltpu.CompilerParams(collective_id=0),
    )(perm, ids, counts, starts, E.reshape(V_PER * R, 128))
    return out.reshape(T, D)
